# Initial kernel scaffold; baseline (speedup 1.0000x reference)
import jax
import jax.numpy as jnp
from jax.experimental import pallas as pl


def kernel(x, edge_index, edge_attr, W1, b1, W2, b2, dp_W1, dp_b1, dp_W2, dp_b2, W_out, b_out):
    raise NotImplementedError("write your pallas kernel here")


# speedup vs baseline: 4.3953x; 4.3953x over previous
import functools

import jax
import jax.numpy as jnp
from jax import lax
from jax.experimental import pallas as pl
from jax.experimental.pallas import tpu as pltpu
from jax.experimental.pallas import tpu_sc as plsc

N = 10000
E = 320000
D = 128
NC = 2
NS = 16
NPH = 5120
EPT = E // NS
EPP = 20480
CH = 128
NCH = EPP // CH
RPT = NPH // NS
BR = 1000


def _remap_half(dst_v, ew_v, lo):
    def body(r, carry):
        for j in range(CH // 16):
            sl = pl.ds(j * 16, 16)
            d = dst_v[r, sl]
            w = ew_v[r, sl]
            m = (d >= lo) & (d < lo + NPH)
            dst_v[r, sl] = jnp.where(m, d - lo, 0)
            ew_v[r, sl] = jnp.where(m, w, 0.0)
        return carry
    lax.fori_loop(0, NCH, body, 0)


def _make_deg_kernel():
    mesh = plsc.VectorSubcoreMesh(core_axis_name="c", subcore_axis_name="s")

    @functools.partial(
        pl.kernel,
        out_type=jax.ShapeDtypeStruct((NC * NPH,), jnp.float32),
        mesh=mesh,
        scratch_types=[
            pltpu.VMEM((NCH, CH), jnp.int32),
            pltpu.VMEM((NCH, CH), jnp.float32),
            pltpu.VMEM((NPH,), jnp.float32),
            pltpu.VMEM_SHARED((NPH,), jnp.float32),
        ],
    )
    def deg_kernel(dst_hbm, ew_hbm, out_hbm, dst_v, ew_v, stage_v, deg_sh):
        c_id = lax.axis_index("c")
        s_id = lax.axis_index("s")
        lo = c_id * NPH
        pltpu.sync_copy(dst_hbm.at[s_id], dst_v)
        pltpu.sync_copy(ew_hbm.at[s_id], ew_v)
        _remap_half(dst_v, ew_v, lo)

        @pl.when(s_id == 0)
        def _zero():
            def zbody(i, carry):
                stage_v[pl.ds(i * 16, 16)] = jnp.zeros((16,), jnp.float32)
                return carry
            lax.fori_loop(0, NPH // 16, zbody, 0)
            pltpu.sync_copy(stage_v, deg_sh)

        plsc.subcore_barrier()

        def chunk(c, carry):
            pltpu.sync_copy(ew_v.at[c], deg_sh.at[dst_v.at[c]], add=True)
            return carry
        lax.fori_loop(0, NCH, chunk, 0)

        plsc.subcore_barrier()

        @pl.when(s_id == 0)
        def _out():
            pltpu.sync_copy(deg_sh, stage_v)
            pltpu.sync_copy(stage_v, out_hbm.at[pl.ds(c_id * NPH, NPH)])

    return deg_kernel


def _make_agg_kernel():
    mesh = plsc.VectorSubcoreMesh(core_axis_name="c", subcore_axis_name="s")

    @functools.partial(
        pl.kernel,
        out_type=jax.ShapeDtypeStruct((NC, NPH, D), jnp.float32),
        mesh=mesh,
        scratch_types=[
            pltpu.VMEM((NCH, CH), jnp.int32),
            pltpu.VMEM((NCH, CH), jnp.int32),
            pltpu.VMEM((NCH, CH), jnp.float32),
            pltpu.VMEM((CH, D), jnp.float32),
            pltpu.VMEM((64, D), jnp.float32),
            pltpu.VMEM_SHARED((NPH, D), jnp.float32),
            pltpu.SemaphoreType.DMA,
        ],
    )
    def agg_kernel(xs_hbm, src_hbm, dst_hbm, ew_hbm, out_hbm,
                   src_v, dst_v, ew_v, rows_v, zero_v, acc_sh, sem):
        c_id = lax.axis_index("c")
        s_id = lax.axis_index("s")
        lo = c_id * NPH
        pltpu.sync_copy(src_hbm.at[s_id], src_v)
        pltpu.sync_copy(dst_hbm.at[s_id], dst_v)
        pltpu.sync_copy(ew_hbm.at[s_id], ew_v)
        _remap_half(dst_v, ew_v, lo)

        def zbody(r, carry):
            for j in range(8):
                zero_v[r, pl.ds(j * 16, 16)] = jnp.zeros((16,), jnp.float32)
            return carry
        lax.fori_loop(0, 64, zbody, 0)
        for k in range(RPT // 64):
            pltpu.sync_copy(zero_v, acc_sh.at[pl.ds(s_id * RPT + k * 64, 64)])

        plsc.subcore_barrier()

        def chunk(c, carry):
            pltpu.async_copy(xs_hbm.at[src_v.at[c]], rows_v, sem).wait()

            def grp(g, gcarry):
                ewv = ew_v[c, pl.ds(g * 16, 16)]
                for r2 in range(16):
                    r = g * 16 + r2
                    s = ewv[r2]
                    for j in range(8):
                        rows_v[r, pl.ds(j * 16, 16)] = (
                            rows_v[r, pl.ds(j * 16, 16)] * s)
                return gcarry
            lax.fori_loop(0, CH // 16, grp, 0)

            pltpu.sync_copy(rows_v, acc_sh.at[dst_v.at[c]], add=True)
            return carry
        lax.fori_loop(0, NCH, chunk, 0)

        plsc.subcore_barrier()

        for k in range(RPT // 64):
            sl = pl.ds(s_id * RPT + k * 64, 64)
            pltpu.sync_copy(acc_sh.at[sl], zero_v)
            pltpu.sync_copy(zero_v, out_hbm.at[c_id].at[sl])

    return agg_kernel


_deg_call = _make_deg_kernel()
_agg_call = _make_agg_kernel()


def _edge_mlp_body(ea_ref, w1_ref, b1_ref, w2_ref, b2_ref, out_ref):
    a = ea_ref[:]
    acc = jnp.zeros_like(a) + b2_ref[0, 0]
    for j in range(16):
        h = jnp.maximum(a * w1_ref[0, j] + b1_ref[0, j], 0.0)
        acc = acc + h * w2_ref[0, j]
    out_ref[:] = 1.0 / (1.0 + jnp.exp(-acc))


def _edge_mlp(ea, w1, b1, w2, b2):
    return pl.pallas_call(
        _edge_mlp_body,
        out_shape=jax.ShapeDtypeStruct(ea.shape, jnp.float32),
    )(ea, w1, b1, w2, b2)


def _scale_body(x_ref, w_ref, deg_ref, xs_ref, dis_ref):
    xw = jnp.dot(x_ref[:], w_ref[:], preferred_element_type=jnp.float32)
    dis = lax.rsqrt(deg_ref[:] + 1.0)
    xs_ref[:] = xw * dis
    dis_ref[:] = dis


def _first_scale(x, w1, deg_col):
    return pl.pallas_call(
        _scale_body,
        grid=(N // BR,),
        in_specs=[
            pl.BlockSpec((BR, D), lambda i: (i, 0)),
            pl.BlockSpec((D, D), lambda i: (0, 0)),
            pl.BlockSpec((BR, 1), lambda i: (i, 0)),
        ],
        out_specs=[
            pl.BlockSpec((BR, D), lambda i: (i, 0)),
            pl.BlockSpec((BR, 1), lambda i: (i, 0)),
        ],
        out_shape=[
            jax.ShapeDtypeStruct((N, D), jnp.float32),
            jax.ShapeDtypeStruct((N, 1), jnp.float32),
        ],
    )(x, w1, deg_col)


def _combine_body(agg_ref, xs_ref, dis_ref, w_ref, bpre_ref, gate_ref,
                  scol_ref, cpost_ref, out_ref):
    t = (agg_ref[:] + xs_ref[:]) * dis_ref[:]
    u = t + bpre_ref[:]
    h = jnp.maximum(u, gate_ref[0, 0] * u)
    out_ref[:] = (jnp.dot(h, w_ref[:], preferred_element_type=jnp.float32)
                  * scol_ref[:] + cpost_ref[:])


def _combine(agg, xs, dis, w, bpre, gate, scol, cpost):
    return pl.pallas_call(
        _combine_body,
        grid=(N // BR,),
        in_specs=[
            pl.BlockSpec((BR, D), lambda i: (i, 0)),
            pl.BlockSpec((BR, D), lambda i: (i, 0)),
            pl.BlockSpec((BR, 1), lambda i: (i, 0)),
            pl.BlockSpec((D, D), lambda i: (0, 0)),
            pl.BlockSpec((1, D), lambda i: (0, 0)),
            pl.BlockSpec((1, 1), lambda i: (0, 0)),
            pl.BlockSpec((BR, 1), lambda i: (i, 0)),
            pl.BlockSpec((1, D), lambda i: (0, 0)),
        ],
        out_specs=pl.BlockSpec((BR, D), lambda i: (i, 0)),
        out_shape=jax.ShapeDtypeStruct((N, D), jnp.float32),
    )(agg, xs, dis, w, bpre, gate, scol, cpost)


def kernel(x, edge_index, edge_attr, W1, b1, W2, b2, dp_W1, dp_b1, dp_W2,
           dp_b2, W_out, b_out):
    def _tile_pad(a):
        a = a.reshape(NS, EPT)
        a = jnp.pad(a, ((0, 0), (0, EPP - EPT)))
        return a.reshape(NS, NCH, CH)

    src_r = _tile_pad(edge_index[0])
    dst_r = _tile_pad(edge_index[1])

    ea = edge_attr.reshape(E // D, D)
    ew = _edge_mlp(ea, dp_W1.reshape(1, 16), dp_b1.reshape(1, 16),
                   dp_W2.reshape(1, 16), dp_b2.reshape(1, 1))
    ew_r = _tile_pad(ew.reshape(E))

    degp = _deg_call(dst_r, ew_r)
    deg_col = degp[:N].reshape(N, 1)
    xs1, dis = _first_scale(x, W1, deg_col)

    stack_w = jnp.stack([W2, W_out])
    stack_bpre = jnp.stack([b1.reshape(1, D), b2.reshape(1, D)])
    stack_gate = jnp.array([0.0, 1.0], jnp.float32).reshape(2, 1, 1)
    stack_scol = jnp.stack([dis, jnp.ones_like(dis)])
    stack_cpost = jnp.stack([jnp.zeros((1, D), jnp.float32),
                             b_out.reshape(1, D)])

    def body(xs, per):
        w, bpre, gate, scol, cpost = per
        agg = _agg_call(xs, src_r, dst_r, ew_r).reshape(NC * NPH, D)[:N]
        y = _combine(agg, xs, dis, w, bpre, gate, scol, cpost)
        return y, None

    out, _ = lax.scan(
        body, xs1, (stack_w, stack_bpre, stack_gate, stack_scol, stack_cpost))
    return out


# speedup vs baseline: 4.8986x; 1.1145x over previous
import functools

import jax
import jax.numpy as jnp
from jax import lax
from jax.experimental import pallas as pl
from jax.experimental.pallas import tpu as pltpu
from jax.experimental.pallas import tpu_sc as plsc

N = 10000
E = 320000
D = 128
NC = 2
NS = 16
NPH = 5120
EPT = E // NS
EPP = 20480
CH = 128
NCH = EPP // CH
RPT = NPH // NS
SB = 16
NSB = NCH // SB
BR = 1000


def _remap_half(dst_v, ew_v, lo):
    def body(r, carry):
        for j in range(CH // 16):
            sl = pl.ds(j * 16, 16)
            d = dst_v[r, sl]
            w = ew_v[r, sl]
            m = (d >= lo) & (d < lo + NPH)
            dst_v[r, sl] = jnp.where(m, d - lo, 0)
            ew_v[r, sl] = jnp.where(m, w, 0.0)
        return carry
    lax.fori_loop(0, NCH, body, 0)


def _make_deg_kernel():
    mesh = plsc.VectorSubcoreMesh(core_axis_name="c", subcore_axis_name="s")

    @functools.partial(
        pl.kernel,
        out_type=jax.ShapeDtypeStruct((NC * NPH,), jnp.float32),
        mesh=mesh,
        scratch_types=[
            pltpu.VMEM((NCH, CH), jnp.int32),
            pltpu.VMEM((NCH, CH), jnp.float32),
            pltpu.VMEM((NPH,), jnp.float32),
            pltpu.VMEM_SHARED((NPH,), jnp.float32),
        ],
    )
    def deg_kernel(dst_hbm, ew_hbm, out_hbm, dst_v, ew_v, stage_v, deg_sh):
        c_id = lax.axis_index("c")
        s_id = lax.axis_index("s")
        lo = c_id * NPH
        pltpu.sync_copy(dst_hbm.at[s_id], dst_v)
        pltpu.sync_copy(ew_hbm.at[s_id], ew_v)
        _remap_half(dst_v, ew_v, lo)

        @pl.when(s_id == 0)
        def _zero():
            def zbody(i, carry):
                stage_v[pl.ds(i * 16, 16)] = jnp.zeros((16,), jnp.float32)
                return carry
            lax.fori_loop(0, NPH // 16, zbody, 0)
            pltpu.sync_copy(stage_v, deg_sh)

        plsc.subcore_barrier()

        def chunk(c, carry):
            pltpu.sync_copy(ew_v.at[c], deg_sh.at[dst_v.at[c]], add=True)
            return carry
        lax.fori_loop(0, NCH, chunk, 0)

        plsc.subcore_barrier()

        @pl.when(s_id == 0)
        def _out():
            pltpu.sync_copy(deg_sh, stage_v)
            pltpu.sync_copy(stage_v, out_hbm.at[pl.ds(c_id * NPH, NPH)])

    return deg_kernel


def _make_agg_kernel():
    mesh = plsc.VectorSubcoreMesh(core_axis_name="c", subcore_axis_name="s")

    @functools.partial(
        pl.kernel,
        out_type=jax.ShapeDtypeStruct((NC, NPH, D), jnp.float32),
        mesh=mesh,
        scratch_types=[
            pltpu.VMEM((2, SB, CH), jnp.int32),
            pltpu.VMEM((2, SB, CH), jnp.int32),
            pltpu.VMEM((2, SB, CH), jnp.float32),
            pltpu.VMEM((CH, D), jnp.float32),
            pltpu.VMEM((CH, D), jnp.float32),
            pltpu.VMEM((CH, D), jnp.float32),
            pltpu.VMEM((CH, D), jnp.float32),
            pltpu.VMEM((64, D), jnp.float32),
            pltpu.VMEM_SHARED((NPH, D), jnp.float32),
            [pltpu.SemaphoreType.DMA] * 4,
            [pltpu.SemaphoreType.DMA] * 4,
            pltpu.SemaphoreType.DMA,
            pltpu.SemaphoreType.DMA,
        ],
    )
    def agg_kernel(xs_hbm, src_hbm, dst_hbm, ew_hbm, out_hbm,
                   srcb_v, dstb_v, ewb_v, r0, r1, r2, r3, zero_v, acc_sh,
                   gsems, ssems, isem, srcsem):
        rows = [r0, r1, r2, r3]
        c_id = lax.axis_index("c")
        s_id = lax.axis_index("s")
        lo = c_id * NPH
        pltpu.sync_copy(src_hbm.at[s_id].at[pl.ds(0, SB)], srcb_v.at[0])
        pltpu.async_copy(dst_hbm.at[s_id].at[pl.ds(0, SB)], dstb_v.at[0], isem)
        pltpu.async_copy(ew_hbm.at[s_id].at[pl.ds(0, SB)], ewb_v.at[0], isem)
        for b in range(2):
            pltpu.async_copy(xs_hbm.at[srcb_v.at[0, b]], rows[b], gsems[b])

        def zbody(r, carry):
            for j in range(8):
                zero_v[r, pl.ds(j * 16, 16)] = jnp.zeros((16,), jnp.float32)
            return carry
        lax.fori_loop(0, 64, zbody, 0)
        for k in range(RPT // 64):
            pltpu.sync_copy(zero_v, acc_sh.at[pl.ds(s_id * RPT + k * 64, 64)])

        plsc.subcore_barrier()

        def _wait_rows(sem):
            pltpu.make_async_copy(xs_hbm.at[srcb_v.at[0, 0]], r0, sem).wait()

        def _wait_blk(ref, sem):
            pltpu.make_async_copy(dst_hbm.at[s_id].at[pl.ds(0, SB)],
                                  ref, sem).wait()

        def _scale(buf, p, c2):
            def grp(g, gcarry):
                ewv = ewb_v[p, c2, pl.ds(g * 16, 16)]
                for r2 in range(16):
                    r = g * 16 + r2
                    s = ewv[r2]
                    for j in range(8):
                        buf[r, pl.ds(j * 16, 16)] = buf[r, pl.ds(j * 16, 16)] * s
                return gcarry
            lax.fori_loop(0, CH // 16, grp, 0)

        def block(sb, carry):
            p = lax.rem(sb, 2)
            q = 1 - p
            _wait_blk(dstb_v.at[p], isem)
            _wait_blk(ewb_v.at[p], isem)

            def rbody(r, rcarry):
                for j in range(CH // 16):
                    sl = pl.ds(j * 16, 16)
                    d = dstb_v[p, r, sl]
                    w = ewb_v[p, r, sl]
                    m = (d >= lo) & (d < lo + NPH)
                    dstb_v[p, r, sl] = jnp.where(m, d - lo, 0)
                    ewb_v[p, r, sl] = jnp.where(m, w, 0.0)
                return rcarry
            lax.fori_loop(0, SB, rbody, 0)

            for c2 in range(SB):
                b = c2 % 4
                b2 = (c2 + 2) % 4
                c = sb * SB + c2
                _wait_rows(gsems[b])
                _scale(rows[b], p, c2)

                @pl.when(c > 1)
                def _():
                    _wait_rows(ssems[b2])

                if c2 == 1:
                    @pl.when(sb + 1 < NSB)
                    def _():
                        nb = pl.ds((sb + 1) * SB, SB)
                        pltpu.async_copy(src_hbm.at[s_id].at[nb],
                                         srcb_v.at[q], srcsem)
                        pltpu.async_copy(dst_hbm.at[s_id].at[nb],
                                         dstb_v.at[q], isem)
                        pltpu.async_copy(ew_hbm.at[s_id].at[nb],
                                         ewb_v.at[q], isem)

                if c2 == 13:
                    @pl.when(sb + 1 < NSB)
                    def _():
                        _wait_blk(srcb_v.at[q], srcsem)

                if c2 + 2 < SB:
                    pltpu.async_copy(xs_hbm.at[srcb_v.at[p, c2 + 2]],
                                     rows[b2], gsems[b2])
                else:
                    @pl.when(sb + 1 < NSB)
                    def _():
                        pltpu.async_copy(xs_hbm.at[srcb_v.at[q, c2 + 2 - SB]],
                                         rows[b2], gsems[b2])
                pltpu.async_copy(rows[b], acc_sh.at[dstb_v.at[p, c2]],
                                 ssems[b], add=True)
            return carry
        lax.fori_loop(0, NSB, block, 0)
        _wait_rows(ssems[(NCH - 2) % 4])
        _wait_rows(ssems[(NCH - 1) % 4])

        plsc.subcore_barrier()

        for k in range(RPT // 64):
            sl = pl.ds(s_id * RPT + k * 64, 64)
            pltpu.sync_copy(acc_sh.at[sl], zero_v)
            pltpu.sync_copy(zero_v, out_hbm.at[c_id].at[sl])

    return agg_kernel


_deg_call = _make_deg_kernel()
_agg_call = _make_agg_kernel()


def _edge_mlp_body(ea_ref, w1_ref, b1_ref, w2_ref, b2_ref, out_ref):
    a = ea_ref[:]
    acc = jnp.zeros_like(a) + b2_ref[0, 0]
    for j in range(16):
        h = jnp.maximum(a * w1_ref[0, j] + b1_ref[0, j], 0.0)
        acc = acc + h * w2_ref[0, j]
    out_ref[:] = 1.0 / (1.0 + jnp.exp(-acc))


def _edge_mlp(ea, w1, b1, w2, b2):
    return pl.pallas_call(
        _edge_mlp_body,
        out_shape=jax.ShapeDtypeStruct(ea.shape, jnp.float32),
    )(ea, w1, b1, w2, b2)


def _scale_body(x_ref, w_ref, deg_ref, xs_ref, dis_ref):
    xw = jnp.dot(x_ref[:], w_ref[:], preferred_element_type=jnp.float32)
    dis = lax.rsqrt(deg_ref[:] + 1.0)
    xs_ref[:] = xw * dis
    dis_ref[:] = dis


def _first_scale(x, w1, deg_col):
    return pl.pallas_call(
        _scale_body,
        grid=(N // BR,),
        in_specs=[
            pl.BlockSpec((BR, D), lambda i: (i, 0)),
            pl.BlockSpec((D, D), lambda i: (0, 0)),
            pl.BlockSpec((BR, 1), lambda i: (i, 0)),
        ],
        out_specs=[
            pl.BlockSpec((BR, D), lambda i: (i, 0)),
            pl.BlockSpec((BR, 1), lambda i: (i, 0)),
        ],
        out_shape=[
            jax.ShapeDtypeStruct((N, D), jnp.float32),
            jax.ShapeDtypeStruct((N, 1), jnp.float32),
        ],
    )(x, w1, deg_col)


def _combine_body(agg_ref, xs_ref, dis_ref, w_ref, bpre_ref, gate_ref,
                  scol_ref, cpost_ref, out_ref):
    t = (agg_ref[:] + xs_ref[:]) * dis_ref[:]
    u = t + bpre_ref[:]
    h = jnp.maximum(u, gate_ref[0, 0] * u)
    out_ref[:] = (jnp.dot(h, w_ref[:], preferred_element_type=jnp.float32)
                  * scol_ref[:] + cpost_ref[:])


def _combine(agg, xs, dis, w, bpre, gate, scol, cpost):
    return pl.pallas_call(
        _combine_body,
        grid=(N // BR,),
        in_specs=[
            pl.BlockSpec((BR, D), lambda i: (i, 0)),
            pl.BlockSpec((BR, D), lambda i: (i, 0)),
            pl.BlockSpec((BR, 1), lambda i: (i, 0)),
            pl.BlockSpec((D, D), lambda i: (0, 0)),
            pl.BlockSpec((1, D), lambda i: (0, 0)),
            pl.BlockSpec((1, 1), lambda i: (0, 0)),
            pl.BlockSpec((BR, 1), lambda i: (i, 0)),
            pl.BlockSpec((1, D), lambda i: (0, 0)),
        ],
        out_specs=pl.BlockSpec((BR, D), lambda i: (i, 0)),
        out_shape=jax.ShapeDtypeStruct((N, D), jnp.float32),
    )(agg, xs, dis, w, bpre, gate, scol, cpost)


def kernel(x, edge_index, edge_attr, W1, b1, W2, b2, dp_W1, dp_b1, dp_W2,
           dp_b2, W_out, b_out):
    def _tile_pad(a):
        a = a.reshape(NS, EPT)
        a = jnp.pad(a, ((0, 0), (0, EPP - EPT)))
        return a.reshape(NS, NCH, CH)

    src_r = _tile_pad(edge_index[0])
    dst_r = _tile_pad(edge_index[1])

    ea = edge_attr.reshape(E // D, D)
    ew = _edge_mlp(ea, dp_W1.reshape(1, 16), dp_b1.reshape(1, 16),
                   dp_W2.reshape(1, 16), dp_b2.reshape(1, 1))
    ew_r = _tile_pad(ew.reshape(E))

    degp = _deg_call(dst_r, ew_r)
    deg_col = degp[:N].reshape(N, 1)
    xs1, dis = _first_scale(x, W1, deg_col)

    stack_w = jnp.stack([W2, W_out])
    stack_bpre = jnp.stack([b1.reshape(1, D), b2.reshape(1, D)])
    stack_gate = jnp.array([0.0, 1.0], jnp.float32).reshape(2, 1, 1)
    stack_scol = jnp.stack([dis, jnp.ones_like(dis)])
    stack_cpost = jnp.stack([jnp.zeros((1, D), jnp.float32),
                             b_out.reshape(1, D)])

    def body(xs, per):
        w, bpre, gate, scol, cpost = per
        agg = _agg_call(xs, src_r, dst_r, ew_r).reshape(NC * NPH, D)[:N]
        y = _combine(agg, xs, dis, w, bpre, gate, scol, cpost)
        return y, None

    out, _ = lax.scan(
        body, xs1, (stack_w, stack_bpre, stack_gate, stack_scol, stack_cpost))
    return out
